# Initial kernel scaffold; baseline (speedup 1.0000x reference)
import jax
import jax.numpy as jnp
from jax.experimental import pallas as pl


def kernel(x, maps, L_idx, idx, node2vec, L_W1, L_b1, L_W2, L_b2, W0, b0, Wrest, brest, bn_gamma, bn_beta, fc1_W, fc1_b, fc2_W, fc2_b):
    raise NotImplementedError("write your pallas kernel here")


# speedup vs baseline: 3.1592x; 3.1592x over previous
import functools

import jax
import jax.numpy as jnp
from jax import lax
from jax.experimental import pallas as pl
from jax.experimental.pallas import tpu as pltpu
from jax.experimental.pallas import tpu_sc as plsc

_M = 325
_NN = 16
_P = 24
_EMB = 16
_B = 16
_F = 64
_G = 4
_NL = 10
_SG = 40
_LHID = 256
_E = _M * _NN
_MM = _M * _M
_MMP = 3308 * 32


def _bdot(a, b):
    return jnp.dot(a.astype(jnp.bfloat16), b.astype(jnp.bfloat16),
                   preferred_element_type=jnp.float32)


def _a_body(maps_ref, w1_ref, b1_ref, w2_ref, out_ref):
    h = jnp.tanh(_bdot(maps_ref[...], w1_ref[0]) + b1_ref[0, 0])
    hb = h.astype(jnp.bfloat16).astype(jnp.float32)
    w2b = w2_ref[0, 0].astype(jnp.bfloat16).astype(jnp.float32)
    h3 = hb.reshape(_M, _NN, _LHID)
    c = jnp.sum(h3 * w2b[None, None, :], axis=2)
    cmax = jnp.max(c, axis=1, keepdims=True)
    e = jnp.exp(c - cmax)
    a = e / jnp.sum(e, axis=1, keepdims=True)
    out_ref[0] = a


def _build_a(maps, L_W1, L_b1, L_W2):
    return pl.pallas_call(
        _a_body,
        grid=(_SG,),
        in_specs=[
            pl.BlockSpec((_E, 5), lambda s: (0, 0)),
            pl.BlockSpec((1, 5, _LHID), lambda s: (s, 0, 0)),
            pl.BlockSpec((1, 1, _LHID), lambda s: (s, 0, 0)),
            pl.BlockSpec((1, 1, _LHID), lambda s: (s, 0, 0)),
        ],
        out_specs=pl.BlockSpec((1, _M, _NN), lambda s: (s, 0, 0)),
        out_shape=jax.ShapeDtypeStruct((_SG, _M, _NN), jnp.float32),
    )(maps, L_W1, L_b1.reshape(_SG, 1, _LHID), L_W2.reshape(_SG, 1, _LHID))


def _sc_scatter(a_vals, idxm):
    mesh = plsc.VectorSubcoreMesh(core_axis_name="c", subcore_axis_name="s")

    @functools.partial(
        pl.kernel,
        mesh=mesh,
        out_type=jax.ShapeDtypeStruct((_SG, _MMP), jnp.float32),
        compiler_params=pltpu.CompilerParams(needs_layout_passes=False),
        scratch_types=[
            pltpu.VMEM((_E,), jnp.int32),
            pltpu.VMEM((_E,), jnp.float32),
            pltpu.VMEM((_MMP,), jnp.float32),
        ],
    )
    def k(a_hbm, idx_hbm, out_hbm, idx_v, val_v, lbuf):
        wid = lax.axis_index("s") * 2 + lax.axis_index("c")
        pltpu.sync_copy(idx_hbm, idx_v)
        zero16 = jnp.zeros((16,), jnp.float32)
        for rep in range(2):
            g = wid + rep * 32

            @pl.when(g < _SG)
            def _():
                pltpu.sync_copy(a_hbm.at[g], val_v)

                def zbody(j, carry):
                    lbuf[pl.ds(j * 16, 16)] = zero16
                    return carry

                lax.fori_loop(0, _MMP // 16, zbody, 0)

                def sbody(j, carry):
                    iv = idx_v[pl.ds(j * 16, 16)]
                    vv = val_v[pl.ds(j * 16, 16)]
                    plsc.store_scatter(lbuf, [iv], vv)
                    return carry

                lax.fori_loop(0, _E // 16, sbody, 0)
                pltpu.sync_copy(lbuf, out_hbm.at[g])

    return k(a_vals, idxm)


def _net_body(x0_ref, ls_ref, w0_ref, wr_ref, g_ref, be_ref, f1w_ref,
              f1b_ref, f2w_ref, f2b_ref, out_ref):
    x0 = x0_ref[...]
    x = jnp.concatenate([x0, x0, x0, x0], axis=1)
    for i in range(_NL):
        ys = []
        for b in range(_B):
            xb = x[b * _M:(b + 1) * _M]
            t = _bdot(ls_ref[i], xb)
            yb = None
            for j in range(_G):
                w = w0_ref[j] if i == 0 else wr_ref[i - 1, j]
                u = _bdot(t[j * _M:(j + 1) * _M], w)
                yb = u if yb is None else yb + u
            ys.append(yb)
        y = jnp.concatenate(ys, axis=0)
        mu = jnp.mean(y, axis=0)
        d = y - mu
        var = jnp.mean(d * d, axis=0)
        yn = d * (g_ref[i] / jnp.sqrt(var + 1e-5)) + be_ref[i]
        x = jnp.maximum(yn, 0.0)
    h = jnp.maximum(_bdot(x, f1w_ref[...]) + f1b_ref[...], 0.0)
    res = _bdot(h, f2w_ref[...])
    out_ref[...] = res + f2b_ref[0]


def _network(x0, lstack, W0, Wrest, gamma, beta, fc1_W, fc1_b, fc2_W, fc2_b):
    return pl.pallas_call(
        _net_body,
        out_shape=jax.ShapeDtypeStruct((_M * _B, 1), jnp.float32),
    )(x0, lstack, W0, Wrest, gamma, beta, fc1_W, fc1_b, fc2_W, fc2_b)


def kernel(x, maps, L_idx, idx, node2vec, L_W1, L_b1, L_W2, L_b2, W0, b0,
           Wrest, brest, bn_gamma, bn_beta, fc1_W, fc1_b, fc2_W, fc2_b):
    x_t = jnp.transpose(x, (0, 2, 1)).reshape(_B * _M, _P)
    e = node2vec[idx]
    e = jnp.broadcast_to(e[None, :, :], (_B, _M, _EMB)).reshape(_B * _M, _EMB)
    x0 = jnp.concatenate([x_t, e], axis=1)

    ar = jnp.arange(_E, dtype=jnp.int32)
    last = jnp.zeros((_MM,), jnp.int32).at[L_idx].set(ar)
    win = last[L_idx] == ar
    idxm = jnp.where(win, L_idx, _MM + (ar & 127)).astype(jnp.int32)

    a_vals = _build_a(maps, L_W1, L_b1, L_W2).reshape(_SG, _E)

    lp = _sc_scatter(a_vals, idxm)
    lstack = lp[:, :_MM].reshape(_NL, _G * _M, _M)

    out = _network(x0, lstack, W0, Wrest, bn_gamma, bn_beta, fc1_W, fc1_b,
                   fc2_W, fc2_b.reshape(1, 1))
    return out.reshape(_B, _M)


# speedup vs baseline: 3.2334x; 1.0235x over previous
import functools

import jax
import jax.numpy as jnp
from jax import lax
from jax.experimental import pallas as pl
from jax.experimental.pallas import tpu as pltpu
from jax.experimental.pallas import tpu_sc as plsc

_M = 325
_NN = 16
_P = 24
_EMB = 16
_B = 16
_F = 64
_G = 4
_NL = 10
_SG = 40
_LHID = 256
_E = _M * _NN
_MM = _M * _M
_MMP = 3308 * 32


def _bdot(a, b):
    return jnp.dot(a.astype(jnp.bfloat16), b.astype(jnp.bfloat16),
                   preferred_element_type=jnp.float32)


def _a_body(maps_ref, w1_ref, b1_ref, w2_ref, out_ref):
    h = jnp.tanh(_bdot(maps_ref[...], w1_ref[0]) + b1_ref[0, 0])
    hb = h.astype(jnp.bfloat16).astype(jnp.float32)
    w2b = w2_ref[0, 0].astype(jnp.bfloat16).astype(jnp.float32)
    h3 = hb.reshape(_M, _NN, _LHID)
    c = jnp.sum(h3 * w2b[None, None, :], axis=2)
    cmax = jnp.max(c, axis=1, keepdims=True)
    e = jnp.exp(c - cmax)
    a = e / jnp.sum(e, axis=1, keepdims=True)
    out_ref[0] = a


def _build_a(maps, L_W1, L_b1, L_W2):
    return pl.pallas_call(
        _a_body,
        grid=(_SG,),
        in_specs=[
            pl.BlockSpec((_E, 5), lambda s: (0, 0)),
            pl.BlockSpec((1, 5, _LHID), lambda s: (s, 0, 0)),
            pl.BlockSpec((1, 1, _LHID), lambda s: (s, 0, 0)),
            pl.BlockSpec((1, 1, _LHID), lambda s: (s, 0, 0)),
        ],
        out_specs=pl.BlockSpec((1, _M, _NN), lambda s: (s, 0, 0)),
        out_shape=jax.ShapeDtypeStruct((_SG, _M, _NN), jnp.float32),
    )(maps, L_W1, L_b1.reshape(_SG, 1, _LHID), L_W2.reshape(_SG, 1, _LHID))


def _sc_scatter(a_vals, idxm):
    mesh = plsc.VectorSubcoreMesh(core_axis_name="c", subcore_axis_name="s")

    @functools.partial(
        pl.kernel,
        mesh=mesh,
        out_type=jax.ShapeDtypeStruct((_SG, _MMP), jnp.float32),
        compiler_params=pltpu.CompilerParams(needs_layout_passes=False),
        scratch_types=[
            pltpu.VMEM((_E,), jnp.int32),
            pltpu.VMEM((_E,), jnp.float32),
            pltpu.VMEM((_MMP,), jnp.float32),
        ],
    )
    def k(a_hbm, idx_hbm, out_hbm, idx_v, val_v, lbuf):
        wid = lax.axis_index("s") * 2 + lax.axis_index("c")
        pltpu.sync_copy(idx_hbm, idx_v)
        zero16 = jnp.zeros((16,), jnp.float32)
        for rep in range(2):
            g = wid + rep * 32

            @pl.when(g < _SG)
            def _():
                pltpu.sync_copy(a_hbm.at[g], val_v)

                def zbody(j, carry):
                    lbuf[pl.ds(j * 16, 16)] = zero16
                    return carry

                lax.fori_loop(0, _MMP // 16, zbody, 0)

                def sbody(j, carry):
                    iv = idx_v[pl.ds(j * 16, 16)]
                    vv = val_v[pl.ds(j * 16, 16)]
                    plsc.store_scatter(lbuf, [iv], vv)
                    return carry

                lax.fori_loop(0, _E // 16, sbody, 0)
                pltpu.sync_copy(lbuf, out_hbm.at[g])

    return k(a_vals, idxm)


def _net_body(x0_ref, ls_ref, w0_ref, wr_ref, g_ref, be_ref, f1w_ref,
              f1b_ref, f2w_ref, f2b_ref, out_ref):
    x0 = x0_ref[...]
    x = jnp.concatenate([x0, x0, x0, x0], axis=2)
    for i in range(_NL):
        ys = []
        for b in range(_B):
            xb = x[b]
            yb = None
            for j in range(_G):
                t = _bdot(ls_ref[i, j], xb)
                w = w0_ref[j] if i == 0 else wr_ref[i - 1, j]
                u = _bdot(t, w)
                yb = u if yb is None else yb + u
            ys.append(yb)
        y = jnp.stack(ys, axis=0)
        mu = jnp.mean(jnp.mean(y, axis=1), axis=0)
        d = y - mu[None, None, :]
        var = jnp.mean(jnp.mean(d * d, axis=1), axis=0)
        yn = d * (g_ref[i] / jnp.sqrt(var + 1e-5))[None, None, :]
        x = jnp.maximum(yn + be_ref[i][None, None, :], 0.0)
    rs = []
    for b in range(_B):
        hb = jnp.maximum(_bdot(x[b], f1w_ref[...]) + f1b_ref[...], 0.0)
        rs.append(_bdot(hb, f2w_ref[...]) + f2b_ref[0])
    out_ref[...] = jnp.stack(rs, axis=0)


def _network(x0, lstack, W0, Wrest, gamma, beta, fc1_W, fc1_b, fc2_W, fc2_b):
    return pl.pallas_call(
        _net_body,
        out_shape=jax.ShapeDtypeStruct((_B, _M, 1), jnp.float32),
    )(x0, lstack, W0, Wrest, gamma, beta, fc1_W, fc1_b, fc2_W, fc2_b)


def kernel(x, maps, L_idx, idx, node2vec, L_W1, L_b1, L_W2, L_b2, W0, b0,
           Wrest, brest, bn_gamma, bn_beta, fc1_W, fc1_b, fc2_W, fc2_b):
    x_t = jnp.transpose(x, (0, 2, 1))
    e = node2vec[idx]
    e = jnp.broadcast_to(e[None, :, :], (_B, _M, _EMB))
    x0 = jnp.concatenate([x_t, e], axis=2)

    ar = jnp.arange(_E, dtype=jnp.int32)
    last = jnp.zeros((_MM,), jnp.int32).at[L_idx].set(ar)
    win = last[L_idx] == ar
    idxm = jnp.where(win, L_idx, _MM + (ar & 127)).astype(jnp.int32)

    a_vals = _build_a(maps, L_W1, L_b1, L_W2).reshape(_SG, _E)

    lp = _sc_scatter(a_vals, idxm)
    lstack = lp[:, :_MM].reshape(_NL, _G, _M, _M)

    out = _network(x0, lstack, W0, Wrest, bn_gamma, bn_beta, fc1_W, fc1_b,
                   fc2_W, fc2_b.reshape(1, 1))
    return out.reshape(_B, _M)
